# baseline (speedup 1.0000x reference)
import functools

import jax
import jax.numpy as jnp
from jax import lax
from jax.experimental import pallas as pl
from jax.experimental.pallas import tpu as pltpu
from jax.experimental.pallas import tpu_sc as plsc

L = 16
NROWS = 128
DIM = 32768
NCHUNK = DIM // L
NC = 2
NS = 16
NW = NC * NS
ROWS_PER = NROWS // NW
CAP = 16376
FAST_MAX = CAP - L
BISECT_ITERS = 12


_mesh = plsc.VectorSubcoreMesh(
    core_axis_name="c", subcore_axis_name="s", num_cores=NC, num_subcores=NS
)


@functools.partial(
    pl.kernel,
    out_type=jax.ShapeDtypeStruct((NROWS, DIM), jnp.float32),
    mesh=_mesh,
    scratch_types=[
        pltpu.VMEM((DIM,), jnp.float32),
        pltpu.VMEM((DIM,), jnp.float32),
        pltpu.VMEM((DIM,), jnp.float32),
        pltpu.VMEM((CAP,), jnp.float32),
        pltpu.SemaphoreType.DMA,
        pltpu.SemaphoreType.DMA,
        pltpu.SemaphoreType.DMA,
    ],
    compiler_params=pltpu.CompilerParams(needs_layout_passes=False),
)
def _sparsemax_sc(x_hbm, o_hbm, in0, in1, ob, cand_v, s0, s1, so):
    vzero = jnp.zeros((L,), jnp.float32)
    vone_i = jnp.ones((L,), jnp.int32)

    def sampled_max(buf):
        def gstep(g, acc):
            a = jnp.maximum(buf[pl.ds(g * 8 * 2 * L, L)],
                            buf[pl.ds((g * 8 * 2 + 8) * L, L)])
            return jnp.maximum(acc, a)

        acc = plsc.parallel_loop(
            0, NCHUNK // 16, 1, unroll=4,
            carry=jnp.full((L,), -jnp.inf, jnp.float32),
        )(gstep)
        return jnp.max(acc)

    def compress(buf, cutoff):
        def step(i, carry):
            off_m1, mx = carry
            v = buf[pl.ds(i * L, L)]
            mask = v > cutoff
            prefix = plsc.cumsum(vone_i, mask=mask)
            dest = off_m1 + prefix
            dest = plsc.bitcast(
                jnp.minimum(
                    plsc.bitcast(dest, jnp.uint32), jnp.uint32(CAP - 1)
                ),
                jnp.int32,
            )
            plsc.store_scatter(cand_v, [dest], v, mask=mask)
            cnt = plsc.all_reduce_population_count(mask)
            return off_m1 + cnt, jnp.maximum(mx, v)

        off_m1, mx = plsc.parallel_loop(
            0, NCHUNK, 1, unroll=16,
            carry=(
                jnp.full((L,), -1, jnp.int32),
                jnp.full((L,), -jnp.inf, jnp.float32),
            ),
        )(step)
        return off_m1[0] + 1, jnp.max(mx)

    def nchunks(m):
        return (m + (L - 1)) >> 4

    def g3_cand(m, t1, t2, t3):
        def step(i, accs):
            a1, a2, a3 = accs
            v = cand_v[pl.ds(i * L, L)]
            return (
                a1 + jnp.maximum(v - t1, 0.0),
                a2 + jnp.maximum(v - t2, 0.0),
                a3 + jnp.maximum(v - t3, 0.0),
            )

        a1, a2, a3 = plsc.parallel_loop(
            0, nchunks(m), 1, unroll=2, carry=(vzero, vzero, vzero)
        )(step)
        return jnp.sum(a1), jnp.sum(a2), jnp.sum(a3)

    def ks_cand(m, lo):
        def step(i, acc):
            ka, sa = acc
            v = cand_v[pl.ds(i * L, L)]
            sel = v > lo
            return (
                ka + jnp.where(sel, 1.0, 0.0),
                sa + jnp.where(sel, v, 0.0),
            )

        ka, sa = plsc.parallel_loop(
            0, nchunks(m), 1, unroll=2, carry=(vzero, vzero)
        )(step)
        return jnp.sum(ka), jnp.sum(sa)

    def g3_dense(buf, t1, t2, t3):
        def step(i, accs):
            a1, a2, a3 = accs
            v = buf[pl.ds(i * L, L)]
            return (
                a1 + jnp.maximum(v - t1, 0.0),
                a2 + jnp.maximum(v - t2, 0.0),
                a3 + jnp.maximum(v - t3, 0.0),
            )

        a1, a2, a3 = lax.fori_loop(
            0, NCHUNK, step, (vzero, vzero, vzero), unroll=4
        )
        return jnp.sum(a1), jnp.sum(a2), jnp.sum(a3)

    def ks_dense(buf, lo):
        def step(i, acc):
            ka, sa = acc
            v = buf[pl.ds(i * L, L)]
            sel = v > lo
            return (
                ka + jnp.where(sel, 1.0, 0.0),
                sa + jnp.where(sel, v, 0.0),
            )

        ka, sa = lax.fori_loop(0, NCHUNK, step, (vzero, vzero))
        return jnp.sum(ka), jnp.sum(sa)

    def solve(g3_fn, ks_fn, lo0, hi0):
        def bis(_, lh):
            lo, hi = lh
            w = hi - lo
            t1 = lo + 0.25 * w
            t2 = lo + 0.5 * w
            t3 = lo + 0.75 * w
            g1, g2, g3 = g3_fn(t1, t2, t3)
            ge1 = g1 >= 1.0
            ge2 = g2 >= 1.0
            ge3 = g3 >= 1.0
            lo2 = jnp.where(ge1, t1, lo)
            lo2 = jnp.where(ge2, t2, lo2)
            lo2 = jnp.where(ge3, t3, lo2)
            hi2 = jnp.where(ge3, hi, t3)
            hi2 = jnp.where(ge2, hi2, t2)
            hi2 = jnp.where(ge1, hi2, t1)
            return lo2, hi2

        lo, _ = lax.fori_loop(
            0,
            BISECT_ITERS,
            bis,
            (jnp.broadcast_to(lo0, (L,)), jnp.broadcast_to(hi0, (L,))),
        )
        kk, ss = ks_fn(lo)
        return (jnp.broadcast_to(ss, (L,)) - 1.0) / jnp.broadcast_to(kk, (L,))

    def dense_relu(buf, t, lo_chunk, hi_chunk):
        def step(i):
            ob[pl.ds(i * L, L)] = jnp.maximum(buf[pl.ds(i * L, L)] - t, 0.0)

        plsc.parallel_loop(lo_chunk, hi_chunk, 1, unroll=8)(step)

    wid = lax.axis_index("s") * NC + lax.axis_index("c")
    row0 = wid * ROWS_PER
    bufs = (in0, in1)
    sems = (s0, s1)

    in_handles = {0: pltpu.async_copy(x_hbm.at[row0], in0, s0)}
    out_handle = None

    for j in range(ROWS_PER):
        buf = bufs[j % 2]
        with jax.named_scope("iwait"):
            in_handles[j].wait()
        if j + 1 < ROWS_PER:
            in_handles[j + 1] = pltpu.async_copy(
                x_hbm.at[row0 + j + 1], bufs[(j + 1) % 2], sems[(j + 1) % 2]
            )

        with jax.named_scope("smax"):
            smax = sampled_max(buf)
        cut_v = jnp.broadcast_to(smax - 1.0, (L,))
        with jax.named_scope("compress"):
            m, xmax = compress(buf, cut_v)
        cand_v[pl.ds(jnp.minimum(m, CAP - L), L)] = jnp.full(
            (L,), -jnp.inf, jnp.float32
        )
        cutoff = xmax - 1.0
        fast = m <= FAST_MAX

        def fast_fn(opnd):
            mm, cut, xm = opnd
            return solve(
                lambda a, b, c: g3_cand(mm, a, b, c),
                lambda lo: ks_cand(mm, lo),
                cut,
                xm,
            )

        def slow_fn(opnd, buf=buf):
            mm, cut, xm = opnd
            return solve(
                lambda a, b, c: g3_dense(buf, a, b, c),
                lambda lo: ks_dense(buf, lo),
                cut,
                xm,
            )

        with jax.named_scope("solve"):
            t = lax.cond(fast, fast_fn, slow_fn, (m, cutoff, xmax))

        with jax.named_scope("owait"):
            if out_handle is not None:
                out_handle[0].wait()
                out_handle[1].wait()
        with jax.named_scope("relu"):
            dense_relu(buf, t, 0, NCHUNK // 2)
            h0 = pltpu.async_copy(
                ob.at[pl.ds(0, DIM // 2)],
                o_hbm.at[row0 + j, pl.ds(0, DIM // 2)],
                so,
            )
            dense_relu(buf, t, NCHUNK // 2, NCHUNK)
            h1 = pltpu.async_copy(
                ob.at[pl.ds(DIM // 2, DIM // 2)],
                o_hbm.at[row0 + j, pl.ds(DIM // 2, DIM // 2)],
                so,
            )
        out_handle = (h0, h1)

    out_handle[0].wait()
    out_handle[1].wait()


def kernel(x):
    return _sparsemax_sc(x)


# speedup vs baseline: 1.0506x; 1.0506x over previous
import functools

import jax
import jax.numpy as jnp
from jax import lax
from jax.experimental import pallas as pl
from jax.experimental.pallas import tpu as pltpu
from jax.experimental.pallas import tpu_sc as plsc

L = 16
NROWS = 128
DIM = 32768
NCHUNK = DIM // L
NC = 2
NS = 16
NW = NC * NS
ROWS_PER = NROWS // NW
CAP = 16376
FAST_MAX = CAP - L
BISECT_ITERS = 12


_mesh = plsc.VectorSubcoreMesh(
    core_axis_name="c", subcore_axis_name="s", num_cores=NC, num_subcores=NS
)


@functools.partial(
    pl.kernel,
    out_type=jax.ShapeDtypeStruct((NROWS, DIM), jnp.float32),
    mesh=_mesh,
    scratch_types=[
        pltpu.VMEM((DIM,), jnp.float32),
        pltpu.VMEM((DIM,), jnp.float32),
        pltpu.VMEM((DIM,), jnp.float32),
        pltpu.VMEM((CAP,), jnp.float32),
        pltpu.SemaphoreType.DMA,
        pltpu.SemaphoreType.DMA,
        pltpu.SemaphoreType.DMA,
    ],
    compiler_params=pltpu.CompilerParams(needs_layout_passes=False),
)
def _sparsemax_sc(x_hbm, o_hbm, in0, in1, ob, cand_v, s0, s1, so):
    vzero = jnp.zeros((L,), jnp.float32)
    vone_i = jnp.ones((L,), jnp.int32)

    def sampled_max(buf):
        def gstep(g, acc):
            a = jnp.maximum(buf[pl.ds(g * 8 * 2 * L, L)],
                            buf[pl.ds((g * 8 * 2 + 8) * L, L)])
            return jnp.maximum(acc, a)

        acc = plsc.parallel_loop(
            0, NCHUNK // 16, 1, unroll=4,
            carry=jnp.full((L,), -jnp.inf, jnp.float32),
        )(gstep)
        return jnp.max(acc)

    def compress(buf, cutoff):
        def step(i, carry):
            off_m1, mx = carry
            v = buf[pl.ds(i * L, L)]
            mask = v > cutoff
            prefix = plsc.cumsum(vone_i, mask=mask)
            dest = off_m1 + prefix
            dest = plsc.bitcast(
                jnp.minimum(
                    plsc.bitcast(dest, jnp.uint32), jnp.uint32(CAP - 1)
                ),
                jnp.int32,
            )
            plsc.store_scatter(cand_v, [dest], v, mask=mask)
            cnt = plsc.all_reduce_population_count(mask)
            return off_m1 + cnt, jnp.maximum(mx, v)

        off_m1, mx = plsc.parallel_loop(
            0, NCHUNK, 1, unroll=8,
            carry=(
                jnp.full((L,), -1, jnp.int32),
                jnp.full((L,), -jnp.inf, jnp.float32),
            ),
        )(step)
        return off_m1[0] + 1, jnp.max(mx)

    def nchunks(m):
        return (m + (L - 1)) >> 4

    def g3_cand(m, t1, t2, t3):
        def step(i, accs):
            a1, a2, a3 = accs
            v = cand_v[pl.ds(i * L, L)]
            return (
                a1 + jnp.maximum(v - t1, 0.0),
                a2 + jnp.maximum(v - t2, 0.0),
                a3 + jnp.maximum(v - t3, 0.0),
            )

        a1, a2, a3 = plsc.parallel_loop(
            0, nchunks(m), 1, unroll=2, carry=(vzero, vzero, vzero)
        )(step)
        return jnp.sum(a1), jnp.sum(a2), jnp.sum(a3)

    def ks_cand(m, lo):
        def step(i, acc):
            ka, sa = acc
            v = cand_v[pl.ds(i * L, L)]
            sel = v > lo
            return (
                ka + jnp.where(sel, 1.0, 0.0),
                sa + jnp.where(sel, v, 0.0),
            )

        ka, sa = plsc.parallel_loop(
            0, nchunks(m), 1, unroll=2, carry=(vzero, vzero)
        )(step)
        return jnp.sum(ka), jnp.sum(sa)

    def g3_dense(buf, t1, t2, t3):
        def step(i, accs):
            a1, a2, a3 = accs
            v = buf[pl.ds(i * L, L)]
            return (
                a1 + jnp.maximum(v - t1, 0.0),
                a2 + jnp.maximum(v - t2, 0.0),
                a3 + jnp.maximum(v - t3, 0.0),
            )

        a1, a2, a3 = lax.fori_loop(
            0, NCHUNK, step, (vzero, vzero, vzero), unroll=4
        )
        return jnp.sum(a1), jnp.sum(a2), jnp.sum(a3)

    def ks_dense(buf, lo):
        def step(i, acc):
            ka, sa = acc
            v = buf[pl.ds(i * L, L)]
            sel = v > lo
            return (
                ka + jnp.where(sel, 1.0, 0.0),
                sa + jnp.where(sel, v, 0.0),
            )

        ka, sa = lax.fori_loop(0, NCHUNK, step, (vzero, vzero))
        return jnp.sum(ka), jnp.sum(sa)

    def solve(g3_fn, ks_fn, lo0, hi0):
        def bis(_, lh):
            lo, hi = lh
            w = hi - lo
            t1 = lo + 0.25 * w
            t2 = lo + 0.5 * w
            t3 = lo + 0.75 * w
            g1, g2, g3 = g3_fn(t1, t2, t3)
            ge1 = g1 >= 1.0
            ge2 = g2 >= 1.0
            ge3 = g3 >= 1.0
            lo2 = jnp.where(ge1, t1, lo)
            lo2 = jnp.where(ge2, t2, lo2)
            lo2 = jnp.where(ge3, t3, lo2)
            hi2 = jnp.where(ge3, hi, t3)
            hi2 = jnp.where(ge2, hi2, t2)
            hi2 = jnp.where(ge1, hi2, t1)
            return lo2, hi2

        lo, _ = lax.fori_loop(
            0,
            BISECT_ITERS,
            bis,
            (jnp.broadcast_to(lo0, (L,)), jnp.broadcast_to(hi0, (L,))),
        )
        kk, ss = ks_fn(lo)
        return (jnp.broadcast_to(ss, (L,)) - 1.0) / jnp.broadcast_to(kk, (L,))

    def dense_relu(buf, t, lo_chunk, hi_chunk):
        def step(i):
            ob[pl.ds(i * L, L)] = jnp.maximum(buf[pl.ds(i * L, L)] - t, 0.0)

        plsc.parallel_loop(lo_chunk, hi_chunk, 1, unroll=8)(step)

    wid = lax.axis_index("s") * NC + lax.axis_index("c")
    row0 = wid * ROWS_PER
    bufs = (in0, in1)
    sems = (s0, s1)

    in_handles = {0: pltpu.async_copy(x_hbm.at[row0], in0, s0)}
    out_handle = None

    for j in range(ROWS_PER):
        buf = bufs[j % 2]
        with jax.named_scope("iwait"):
            in_handles[j].wait()
        if j + 1 < ROWS_PER:
            in_handles[j + 1] = pltpu.async_copy(
                x_hbm.at[row0 + j + 1], bufs[(j + 1) % 2], sems[(j + 1) % 2]
            )

        with jax.named_scope("smax"):
            smax = sampled_max(buf)
        cut_v = jnp.broadcast_to(smax - 1.0, (L,))
        with jax.named_scope("compress"):
            m, xmax = compress(buf, cut_v)
        cand_v[pl.ds(jnp.minimum(m, CAP - L), L)] = jnp.full(
            (L,), -jnp.inf, jnp.float32
        )
        cutoff = xmax - 1.0
        fast = m <= FAST_MAX

        def fast_fn(opnd):
            mm, cut, xm = opnd
            return solve(
                lambda a, b, c: g3_cand(mm, a, b, c),
                lambda lo: ks_cand(mm, lo),
                cut,
                xm,
            )

        def slow_fn(opnd, buf=buf):
            mm, cut, xm = opnd
            return solve(
                lambda a, b, c: g3_dense(buf, a, b, c),
                lambda lo: ks_dense(buf, lo),
                cut,
                xm,
            )

        with jax.named_scope("solve"):
            t = lax.cond(fast, fast_fn, slow_fn, (m, cutoff, xmax))

        with jax.named_scope("owait"):
            if out_handle is not None:
                out_handle[0].wait()
                out_handle[1].wait()
        with jax.named_scope("relu"):
            dense_relu(buf, t, 0, NCHUNK // 2)
            h0 = pltpu.async_copy(
                ob.at[pl.ds(0, DIM // 2)],
                o_hbm.at[row0 + j, pl.ds(0, DIM // 2)],
                so,
            )
            dense_relu(buf, t, NCHUNK // 2, NCHUNK)
            h1 = pltpu.async_copy(
                ob.at[pl.ds(DIM // 2, DIM // 2)],
                o_hbm.at[row0 + j, pl.ds(DIM // 2, DIM // 2)],
                so,
            )
        out_handle = (h0, h1)

    out_handle[0].wait()
    out_handle[1].wait()


def kernel(x):
    return _sparsemax_sc(x)
